# baseline (speedup 1.0000x reference)
import functools

import jax
import jax.numpy as jnp
from jax import lax
from jax.experimental import pallas as pl
from jax.experimental.pallas import tpu as pltpu
from jax.experimental.pallas import tpu_sc as plsc

N = 10000
E = 320000
D = 128

_NC = 2
_NS = 16
_CH = 40
_EPT = E // (_NC * _NS)
_NCHUNK = _EPT // _CH
_NBUF = 4
_NG = (_NCHUNK - 2) // _NBUF
_RPB = 624
_RTAIL = N - _NS * _RPB


def _sc_agg_body(x_hbm, src_hbm, dst_hbm, zeros_hbm, out_hbm, agg_sh, *scr):
    cid = lax.axis_index("c")
    sid = lax.axis_index("s")
    ebase = cid * (E // _NC) + sid * _EPT

    k = 0

    def take(nsets):
        nonlocal k
        n = nsets * _NBUF
        out, k = scr[k:k + n], k + n
        return [list(out[s * _NBUF:(s + 1) * _NBUF]) for s in range(nsets)]

    srcv = take(4)
    dstv = take(4)
    isem = take(4)
    rows = take(2)
    dsem = take(2)

    def start_idx(g, i):
        for b in range(_NBUF):
            base = ebase + (g * _NBUF + b) * _CH
            pltpu.async_copy(src_hbm.at[pl.ds(base, _CH)], srcv[i][b], isem[i][b])
            pltpu.async_copy(dst_hbm.at[pl.ds(base, _CH)], dstv[i][b], isem[i][b])

    def wait_idx(i):
        for b in range(_NBUF):
            pltpu.make_async_copy(src_hbm.at[pl.ds(0, _CH)], srcv[i][b], isem[i][b]).wait()
            pltpu.make_async_copy(dst_hbm.at[pl.ds(0, _CH)], dstv[i][b], isem[i][b]).wait()

    def start_gathers(s, i):
        for b in range(_NBUF):
            pltpu.async_copy(x_hbm.at[srcv[i][b]], rows[s][b], dsem[s][b])

    def wait_gather(s, i, b):
        pltpu.make_async_copy(x_hbm.at[srcv[i][b]], rows[s][b], dsem[s][b]).wait()

    def start_scatter(s, i, b):
        pltpu.async_copy(rows[s][b], agg_sh.at[dstv[i][b]], dsem[s][b], add=True)

    def wait_scatter(s, i, b):
        pltpu.make_async_copy(rows[s][b], agg_sh.at[dstv[i][b]], dsem[s][b]).wait()

    row0 = sid * _RPB
    pltpu.sync_copy(zeros_hbm.at[pl.ds(row0, _RPB)], agg_sh.at[pl.ds(row0, _RPB)])

    @pl.when(sid == _NS - 1)
    def _():
        pltpu.sync_copy(zeros_hbm.at[pl.ds(_NS * _RPB, _RTAIL)],
                        agg_sh.at[pl.ds(_NS * _RPB, _RTAIL)])

    start_idx(0, 0)
    start_idx(1, 1)
    wait_idx(0)
    start_gathers(0, 0)

    plsc.subcore_barrier()

    def group_body(g, s, i, first=False):
        o = 1 - s

        def launch_next():
            if not first:
                for b in range(_NBUF):
                    wait_scatter(o, (i + 3) % 4, b)
            wait_idx((i + 1) % 4)
            start_gathers(o, (i + 1) % 4)

        if first:
            launch_next()
        else:
            @pl.when(g + 1 < _NG)
            def _():
                launch_next()

        for b in range(_NBUF):
            wait_gather(s, i, b)
            start_scatter(s, i, b)

        @pl.when(g + 2 < _NG)
        def _():
            start_idx(g + 2, (i + 2) % 4)

    group_body(0, 0, 0, first=True)
    group_body(1, 1, 1)

    def quad(gq, carry):
        g0 = 2 + gq * 4
        group_body(g0, 0, 2)
        group_body(g0 + 1, 1, 3)
        group_body(g0 + 2, 0, 0)
        group_body(g0 + 3, 1, 1)
        return carry

    lax.fori_loop(0, (_NG - 2) // 4, quad, 0)

    for b in range(_NBUF):
        wait_scatter(0, 0, b)
    for b in range(_NBUF):
        wait_scatter(1, 1, b)

    for t in range(_NG * _NBUF, _NCHUNK):
        tbase = ebase + t * _CH
        pltpu.sync_copy(src_hbm.at[pl.ds(tbase, _CH)], srcv[0][0])
        pltpu.sync_copy(dst_hbm.at[pl.ds(tbase, _CH)], dstv[0][0])
        pltpu.async_copy(x_hbm.at[srcv[0][0]], rows[0][0], dsem[0][0]).wait()
        pltpu.sync_copy(rows[0][0], agg_sh.at[dstv[0][0]], add=True)

    plsc.subcore_barrier()

    pltpu.sync_copy(agg_sh.at[pl.ds(row0, _RPB)],
                    out_hbm.at[cid, pl.ds(row0, _RPB)])

    @pl.when(sid == _NS - 1)
    def _():
        pltpu.sync_copy(agg_sh.at[pl.ds(_NS * _RPB, _RTAIL)],
                        out_hbm.at[cid, pl.ds(_NS * _RPB, _RTAIL)])


@functools.partial(jax.jit, static_argnames=())
def _sc_agg(x, src, dst, zeros):
    mesh = plsc.VectorSubcoreMesh(core_axis_name="c", subcore_axis_name="s")
    scratch = (
        [pltpu.VMEM_SHARED((N, D), jnp.float32)]
        + [pltpu.VMEM((_CH,), jnp.int32) for _ in range(4 * _NBUF)]
        + [pltpu.VMEM((_CH,), jnp.int32) for _ in range(4 * _NBUF)]
        + [pltpu.SemaphoreType.DMA for _ in range(4 * _NBUF)]
        + [pltpu.VMEM((_CH, D), jnp.float32) for _ in range(2 * _NBUF)]
        + [pltpu.SemaphoreType.DMA for _ in range(2 * _NBUF)]
    )
    return pl.kernel(
        _sc_agg_body,
        out_type=jax.ShapeDtypeStruct((_NC, N, D), jnp.float32),
        mesh=mesh,
        scratch_types=scratch,
    )(x, src, dst, zeros)


def _tc_mlp_body(eps_ref, x_ref, p_ref, W1_ref, b1_ref, g1_ref, be1_ref,
                 W2_ref, b2_ref, g2_ref, be2_ref, out_ref):
    eps = eps_ref[0, 0]
    h = x_ref[...] * (1.0 + eps) + p_ref[0] + p_ref[1]
    h = jnp.dot(h, W1_ref[...], preferred_element_type=jnp.float32) + b1_ref[...]
    mu = jnp.mean(h, axis=0, keepdims=True)
    var = jnp.mean((h - mu) * (h - mu), axis=0, keepdims=True)
    h = (h - mu) * lax.rsqrt(var + 1e-5) * g1_ref[...] + be1_ref[...]
    h = jnp.maximum(h, 0.0)
    h = jnp.dot(h, W2_ref[...], preferred_element_type=jnp.float32) + b2_ref[...]
    mu = jnp.mean(h, axis=0, keepdims=True)
    var = jnp.mean((h - mu) * (h - mu), axis=0, keepdims=True)
    h = (h - mu) * lax.rsqrt(var + 1e-5) * g2_ref[...] + be2_ref[...]
    out_ref[...] = jnp.maximum(h, 0.0)


def _tc_mlp(eps, x, parts, W1, b1, g1, be1, W2, b2, g2, be2):
    smem = pl.BlockSpec(memory_space=pltpu.SMEM)
    vmem = pl.BlockSpec(memory_space=pltpu.VMEM)
    return pl.pallas_call(
        _tc_mlp_body,
        out_shape=jax.ShapeDtypeStruct((N, D), jnp.float32),
        in_specs=[smem] + [vmem] * 10,
        out_specs=vmem,
    )(eps.reshape(1, 1), x, parts,
      W1, b1.reshape(1, D), g1.reshape(1, D), be1.reshape(1, D),
      W2, b2.reshape(1, D), g2.reshape(1, D), be2.reshape(1, D))


def kernel(adj, x, eps1, W11, b11, g11, be11, W12, b12, g12, be12,
           eps2, W21, b21, g21, be21, W22, b22, g22, be22):
    src = adj[0]
    dst = adj[1]
    zeros = jnp.zeros((N, D), jnp.float32)
    p1 = _sc_agg(x, src, dst, zeros)
    h = _tc_mlp(eps1, x, p1, W11, b11, g11, be11, W12, b12, g12, be12)
    p2 = _sc_agg(h, src, dst, zeros)
    h = _tc_mlp(eps2, h, p2, W21, b21, g21, be21, W22, b22, g22, be22)
    return h


# speedup vs baseline: 1.0337x; 1.0337x over previous
import functools

import jax
import jax.numpy as jnp
from jax import lax
from jax.experimental import pallas as pl
from jax.experimental.pallas import tpu as pltpu
from jax.experimental.pallas import tpu_sc as plsc

N = 10000
E = 320000
D = 128

_NC = 2
_NS = 16
_CH = 80
_EPT = E // (_NC * _NS)
_NCHUNK = _EPT // _CH
_NBUF = 2
_NG = (_NCHUNK - 1) // _NBUF
_RPB = 624
_RTAIL = N - _NS * _RPB


def _sc_agg_body(x_hbm, src_hbm, dst_hbm, zeros_hbm, out_hbm, agg_sh, *scr):
    cid = lax.axis_index("c")
    sid = lax.axis_index("s")
    ebase = cid * (E // _NC) + sid * _EPT

    k = 0

    def take(nsets):
        nonlocal k
        n = nsets * _NBUF
        out, k = scr[k:k + n], k + n
        return [list(out[s * _NBUF:(s + 1) * _NBUF]) for s in range(nsets)]

    srcv = take(4)
    dstv = take(4)
    isem = take(4)
    rows = take(2)
    dsem = take(2)

    def start_idx(g, i):
        for b in range(_NBUF):
            base = ebase + (g * _NBUF + b) * _CH
            pltpu.async_copy(src_hbm.at[pl.ds(base, _CH)], srcv[i][b], isem[i][b])
            pltpu.async_copy(dst_hbm.at[pl.ds(base, _CH)], dstv[i][b], isem[i][b])

    def wait_idx(i):
        for b in range(_NBUF):
            pltpu.make_async_copy(src_hbm.at[pl.ds(0, _CH)], srcv[i][b], isem[i][b]).wait()
            pltpu.make_async_copy(dst_hbm.at[pl.ds(0, _CH)], dstv[i][b], isem[i][b]).wait()

    def start_gathers(s, i):
        for b in range(_NBUF):
            pltpu.async_copy(x_hbm.at[srcv[i][b]], rows[s][b], dsem[s][b])

    def wait_gather(s, i, b):
        pltpu.make_async_copy(x_hbm.at[srcv[i][b]], rows[s][b], dsem[s][b]).wait()

    def start_scatter(s, i, b):
        pltpu.async_copy(rows[s][b], agg_sh.at[dstv[i][b]], dsem[s][b], add=True)

    def wait_scatter(s, i, b):
        pltpu.make_async_copy(rows[s][b], agg_sh.at[dstv[i][b]], dsem[s][b]).wait()

    row0 = sid * _RPB
    pltpu.sync_copy(zeros_hbm.at[pl.ds(row0, _RPB)], agg_sh.at[pl.ds(row0, _RPB)])

    @pl.when(sid == _NS - 1)
    def _():
        pltpu.sync_copy(zeros_hbm.at[pl.ds(_NS * _RPB, _RTAIL)],
                        agg_sh.at[pl.ds(_NS * _RPB, _RTAIL)])

    start_idx(0, 0)
    start_idx(1, 1)
    wait_idx(0)
    start_gathers(0, 0)

    plsc.subcore_barrier()

    def group_body(g, s, i, first=False):
        o = 1 - s

        def launch_next():
            if not first:
                for b in range(_NBUF):
                    wait_scatter(o, (i + 3) % 4, b)
            wait_idx((i + 1) % 4)
            start_gathers(o, (i + 1) % 4)

        if first:
            launch_next()
        else:
            @pl.when(g + 1 < _NG)
            def _():
                launch_next()

        for b in range(_NBUF):
            wait_gather(s, i, b)
            start_scatter(s, i, b)

        @pl.when(g + 2 < _NG)
        def _():
            start_idx(g + 2, (i + 2) % 4)

    group_body(0, 0, 0, first=True)
    group_body(1, 1, 1)

    def quad(gq, carry):
        g0 = 2 + gq * 4
        group_body(g0, 0, 2)
        group_body(g0 + 1, 1, 3)
        group_body(g0 + 2, 0, 0)
        group_body(g0 + 3, 1, 1)
        return carry

    lax.fori_loop(0, (_NG - 2) // 4, quad, 0)

    for b in range(_NBUF):
        wait_scatter(0, 0, b)
    for b in range(_NBUF):
        wait_scatter(1, 1, b)

    tbase = ebase + (_NCHUNK - 1) * _CH
    pltpu.sync_copy(src_hbm.at[pl.ds(tbase, _CH)], srcv[0][0])
    pltpu.sync_copy(dst_hbm.at[pl.ds(tbase, _CH)], dstv[0][0])
    pltpu.async_copy(x_hbm.at[srcv[0][0]], rows[0][0], dsem[0][0]).wait()
    pltpu.sync_copy(rows[0][0], agg_sh.at[dstv[0][0]], add=True)

    plsc.subcore_barrier()

    pltpu.sync_copy(agg_sh.at[pl.ds(row0, _RPB)],
                    out_hbm.at[cid, pl.ds(row0, _RPB)])

    @pl.when(sid == _NS - 1)
    def _():
        pltpu.sync_copy(agg_sh.at[pl.ds(_NS * _RPB, _RTAIL)],
                        out_hbm.at[cid, pl.ds(_NS * _RPB, _RTAIL)])


@functools.partial(jax.jit, static_argnames=())
def _sc_agg(x, src, dst, zeros):
    mesh = plsc.VectorSubcoreMesh(core_axis_name="c", subcore_axis_name="s")
    scratch = (
        [pltpu.VMEM_SHARED((N, D), jnp.float32)]
        + [pltpu.VMEM((_CH,), jnp.int32) for _ in range(4 * _NBUF)]
        + [pltpu.VMEM((_CH,), jnp.int32) for _ in range(4 * _NBUF)]
        + [pltpu.SemaphoreType.DMA for _ in range(4 * _NBUF)]
        + [pltpu.VMEM((_CH, D), jnp.float32) for _ in range(2 * _NBUF)]
        + [pltpu.SemaphoreType.DMA for _ in range(2 * _NBUF)]
    )
    return pl.kernel(
        _sc_agg_body,
        out_type=jax.ShapeDtypeStruct((_NC, N, D), jnp.float32),
        mesh=mesh,
        scratch_types=scratch,
    )(x, src, dst, zeros)


def _tc_mlp_body(eps_ref, x_ref, p_ref, W1_ref, b1_ref, g1_ref, be1_ref,
                 W2_ref, b2_ref, g2_ref, be2_ref, out_ref):
    eps = eps_ref[0, 0]
    h = x_ref[...] * (1.0 + eps) + p_ref[0] + p_ref[1]
    h = jnp.dot(h, W1_ref[...], preferred_element_type=jnp.float32) + b1_ref[...]
    mu = jnp.mean(h, axis=0, keepdims=True)
    var = jnp.mean((h - mu) * (h - mu), axis=0, keepdims=True)
    h = (h - mu) * lax.rsqrt(var + 1e-5) * g1_ref[...] + be1_ref[...]
    h = jnp.maximum(h, 0.0)
    h = jnp.dot(h, W2_ref[...], preferred_element_type=jnp.float32) + b2_ref[...]
    mu = jnp.mean(h, axis=0, keepdims=True)
    var = jnp.mean((h - mu) * (h - mu), axis=0, keepdims=True)
    h = (h - mu) * lax.rsqrt(var + 1e-5) * g2_ref[...] + be2_ref[...]
    out_ref[...] = jnp.maximum(h, 0.0)


def _tc_mlp(eps, x, parts, W1, b1, g1, be1, W2, b2, g2, be2):
    smem = pl.BlockSpec(memory_space=pltpu.SMEM)
    vmem = pl.BlockSpec(memory_space=pltpu.VMEM)
    return pl.pallas_call(
        _tc_mlp_body,
        out_shape=jax.ShapeDtypeStruct((N, D), jnp.float32),
        in_specs=[smem] + [vmem] * 10,
        out_specs=vmem,
    )(eps.reshape(1, 1), x, parts,
      W1, b1.reshape(1, D), g1.reshape(1, D), be1.reshape(1, D),
      W2, b2.reshape(1, D), g2.reshape(1, D), be2.reshape(1, D))


def kernel(adj, x, eps1, W11, b11, g11, be11, W12, b12, g12, be12,
           eps2, W21, b21, g21, be21, W22, b22, g22, be22):
    src = adj[0]
    dst = adj[1]
    zeros = jnp.zeros((N, D), jnp.float32)
    p1 = _sc_agg(x, src, dst, zeros)
    h = _tc_mlp(eps1, x, p1, W11, b11, g11, be11, W12, b12, g12, be12)
    p2 = _sc_agg(h, src, dst, zeros)
    h = _tc_mlp(eps2, h, p2, W21, b21, g21, be21, W22, b22, g22, be22)
    return h


# speedup vs baseline: 1.0731x; 1.0381x over previous
import functools

import jax
import jax.numpy as jnp
from jax import lax
from jax.experimental import pallas as pl
from jax.experimental.pallas import tpu as pltpu
from jax.experimental.pallas import tpu_sc as plsc

N = 10000
E = 320000
D = 128

_NC = 2
_NS = 16
_CH = 80
_EPT = E // (_NC * _NS)
_NCHUNK = _EPT // _CH
_NBUF = 2
_NG = (_NCHUNK - 1) // _NBUF
_RPB = 624
_RTAIL = N - _NS * _RPB


def _sc_agg_body(x_hbm, edges_hbm, zeros_hbm, out_hbm, agg_sh, *scr):
    src_hbm = edges_hbm
    dst_hbm = edges_hbm
    _DOFF = E
    cid = lax.axis_index("c")
    sid = lax.axis_index("s")
    ebase = cid * (E // _NC) + sid * _EPT

    k = 0

    def take(nsets):
        nonlocal k
        n = nsets * _NBUF
        out, k = scr[k:k + n], k + n
        return [list(out[s * _NBUF:(s + 1) * _NBUF]) for s in range(nsets)]

    srcv = take(4)
    dstv = take(4)
    isem = take(4)
    rows = take(2)
    dsem = take(2)

    def start_idx(g, i):
        for b in range(_NBUF):
            base = ebase + (g * _NBUF + b) * _CH
            pltpu.async_copy(src_hbm.at[pl.ds(base, _CH)], srcv[i][b], isem[i][b])
            pltpu.async_copy(dst_hbm.at[pl.ds(E + base, _CH)], dstv[i][b], isem[i][b])

    def wait_idx(i):
        for b in range(_NBUF):
            pltpu.make_async_copy(src_hbm.at[pl.ds(0, _CH)], srcv[i][b], isem[i][b]).wait()
            pltpu.make_async_copy(dst_hbm.at[pl.ds(0, _CH)], dstv[i][b], isem[i][b]).wait()

    def start_gathers(s, i):
        for b in range(_NBUF):
            pltpu.async_copy(x_hbm.at[srcv[i][b]], rows[s][b], dsem[s][b])

    def wait_gather(s, i, b):
        pltpu.make_async_copy(x_hbm.at[srcv[i][b]], rows[s][b], dsem[s][b]).wait()

    def start_scatter(s, i, b):
        pltpu.async_copy(rows[s][b], agg_sh.at[dstv[i][b]], dsem[s][b], add=True)

    def wait_scatter(s, i, b):
        pltpu.make_async_copy(rows[s][b], agg_sh.at[dstv[i][b]], dsem[s][b]).wait()

    row0 = sid * _RPB
    pltpu.sync_copy(zeros_hbm.at[pl.ds(row0, _RPB)], agg_sh.at[pl.ds(row0, _RPB)])

    @pl.when(sid == _NS - 1)
    def _():
        pltpu.sync_copy(zeros_hbm.at[pl.ds(_NS * _RPB, _RTAIL)],
                        agg_sh.at[pl.ds(_NS * _RPB, _RTAIL)])

    start_idx(0, 0)
    start_idx(1, 1)
    wait_idx(0)
    start_gathers(0, 0)

    plsc.subcore_barrier()

    def group_body(g, s, i, first=False):
        o = 1 - s

        def launch_next():
            if not first:
                for b in range(_NBUF):
                    wait_scatter(o, (i + 3) % 4, b)
            wait_idx((i + 1) % 4)
            start_gathers(o, (i + 1) % 4)

        if first:
            launch_next()
        else:
            @pl.when(g + 1 < _NG)
            def _():
                launch_next()

        for b in range(_NBUF):
            wait_gather(s, i, b)
            start_scatter(s, i, b)

        @pl.when(g + 2 < _NG)
        def _():
            start_idx(g + 2, (i + 2) % 4)

    group_body(0, 0, 0, first=True)
    group_body(1, 1, 1)

    def quad(gq, carry):
        g0 = 2 + gq * 4
        group_body(g0, 0, 2)
        group_body(g0 + 1, 1, 3)
        group_body(g0 + 2, 0, 0)
        group_body(g0 + 3, 1, 1)
        return carry

    lax.fori_loop(0, (_NG - 2) // 4, quad, 0)

    for b in range(_NBUF):
        wait_scatter(0, 0, b)
    for b in range(_NBUF):
        wait_scatter(1, 1, b)

    tbase = ebase + (_NCHUNK - 1) * _CH
    pltpu.sync_copy(src_hbm.at[pl.ds(tbase, _CH)], srcv[0][0])
    pltpu.sync_copy(dst_hbm.at[pl.ds(E + tbase, _CH)], dstv[0][0])
    pltpu.async_copy(x_hbm.at[srcv[0][0]], rows[0][0], dsem[0][0]).wait()
    pltpu.sync_copy(rows[0][0], agg_sh.at[dstv[0][0]], add=True)

    plsc.subcore_barrier()

    pltpu.sync_copy(agg_sh.at[pl.ds(row0, _RPB)],
                    out_hbm.at[cid, pl.ds(row0, _RPB)])

    @pl.when(sid == _NS - 1)
    def _():
        pltpu.sync_copy(agg_sh.at[pl.ds(_NS * _RPB, _RTAIL)],
                        out_hbm.at[cid, pl.ds(_NS * _RPB, _RTAIL)])


@functools.partial(jax.jit, static_argnames=())
def _sc_agg(x, edges, zeros):
    mesh = plsc.VectorSubcoreMesh(core_axis_name="c", subcore_axis_name="s")
    scratch = (
        [pltpu.VMEM_SHARED((N, D), jnp.float32)]
        + [pltpu.VMEM((_CH,), jnp.int32) for _ in range(4 * _NBUF)]
        + [pltpu.VMEM((_CH,), jnp.int32) for _ in range(4 * _NBUF)]
        + [pltpu.SemaphoreType.DMA for _ in range(4 * _NBUF)]
        + [pltpu.VMEM((_CH, D), jnp.float32) for _ in range(2 * _NBUF)]
        + [pltpu.SemaphoreType.DMA for _ in range(2 * _NBUF)]
    )
    return pl.kernel(
        _sc_agg_body,
        out_type=jax.ShapeDtypeStruct((_NC, N, D), jnp.float32),
        mesh=mesh,
        scratch_types=scratch,
    )(x, edges, zeros)


def _tc_mlp_body(eps_ref, x_ref, p_ref, W1_ref, b1_ref, g1_ref, be1_ref,
                 W2_ref, b2_ref, g2_ref, be2_ref, out_ref):
    eps = eps_ref[0, 0]
    h = x_ref[...] * (1.0 + eps) + p_ref[0] + p_ref[1]
    h = jnp.dot(h, W1_ref[...], preferred_element_type=jnp.float32) + b1_ref[...]
    mu = jnp.mean(h, axis=0, keepdims=True)
    var = jnp.mean((h - mu) * (h - mu), axis=0, keepdims=True)
    h = (h - mu) * lax.rsqrt(var + 1e-5) * g1_ref[...] + be1_ref[...]
    h = jnp.maximum(h, 0.0)
    h = jnp.dot(h, W2_ref[...], preferred_element_type=jnp.float32) + b2_ref[...]
    mu = jnp.mean(h, axis=0, keepdims=True)
    var = jnp.mean((h - mu) * (h - mu), axis=0, keepdims=True)
    h = (h - mu) * lax.rsqrt(var + 1e-5) * g2_ref[...] + be2_ref[...]
    out_ref[...] = jnp.maximum(h, 0.0)


def _tc_mlp(eps, x, parts, W1, b1, g1, be1, W2, b2, g2, be2):
    smem = pl.BlockSpec(memory_space=pltpu.SMEM)
    vmem = pl.BlockSpec(memory_space=pltpu.VMEM)
    return pl.pallas_call(
        _tc_mlp_body,
        out_shape=jax.ShapeDtypeStruct((N, D), jnp.float32),
        in_specs=[smem] + [vmem] * 10,
        out_specs=vmem,
    )(eps.reshape(1, 1), x, parts,
      W1, b1.reshape(1, D), g1.reshape(1, D), be1.reshape(1, D),
      W2, b2.reshape(1, D), g2.reshape(1, D), be2.reshape(1, D))


def kernel(adj, x, eps1, W11, b11, g11, be11, W12, b12, g12, be12,
           eps2, W21, b21, g21, be21, W22, b22, g22, be22):
    edges = adj.reshape(2 * E)
    zeros = jnp.zeros((N, D), jnp.float32)
    p1 = _sc_agg(x, edges, zeros)
    h = _tc_mlp(eps1, x, p1, W11, b11, g11, be11, W12, b12, g12, be12)
    p2 = _sc_agg(h, edges, zeros)
    h = _tc_mlp(eps2, h, p2, W21, b21, g21, be21, W22, b22, g22, be22)
    return h
